# baseline (speedup 1.0000x reference)
import dataclasses
import functools

import jax
import jax.numpy as jnp
from jax import lax
from jax.experimental import pallas as pl
from jax.experimental.pallas import tpu as pltpu
from jax.experimental.pallas import tpu_sc as plsc

N = 10000
E = 160000
C = 128
BASIS = 128
NL = 4
H = 128
NG = 128
AVG_DEGREE = 23.395238876342773
AVG_NUM_NODES = 77.81317
MAX_RADIUS = 12.0

NC = 2
NS = 16
NW = NC * NS
BLK = 128
NBLK = E // BLK
MAX_BLK_PER_TILE = (NBLK + NW - 1) // NW
NPAD = 10240
ROWS_PER_SUB = NPAD // NS
ZCHUNK = 128

EB = 2000
NB = 1000

_mesh = plsc.VectorSubcoreMesh(core_axis_name="c", subcore_axis_name="s")

_sc_params = pltpu.CompilerParams()
if "needs_layout_passes" in pltpu.CompilerParams.__dataclass_fields__:
    _sc_params = dataclasses.replace(_sc_params, needs_layout_passes=False)


def _tile_id():
    return lax.axis_index("s") * NC + lax.axis_index("c")


def _silu(t):
    return t * (1.0 / (1.0 + jnp.exp(-t)))


@functools.partial(
    pl.kernel,
    out_type=jax.ShapeDtypeStruct((NBLK, BLK), jnp.float32),
    mesh=_mesh,
    scratch_types=[
        pltpu.VMEM((N,), jnp.float32),
        pltpu.VMEM((N,), jnp.float32),
        pltpu.VMEM((N,), jnp.float32),
        pltpu.VMEM((BLK,), jnp.int32),
        pltpu.VMEM((BLK,), jnp.int32),
        pltpu.VMEM((BLK,), jnp.float32),
    ],
    compiler_params=_sc_params,
)
def _sc_d2(px_hbm, py_hbm, pz_hbm, src_hbm, dst_hbm, d2_hbm,
           px_v, py_v, pz_v, srcv, dstv, d2v):
    w = _tile_id()
    pltpu.sync_copy(px_hbm, px_v)
    pltpu.sync_copy(py_hbm, py_v)
    pltpu.sync_copy(pz_hbm, pz_v)

    @pl.loop(0, MAX_BLK_PER_TILE)
    def _(i):
        b = w + i * NW

        @pl.when(b < NBLK)
        def _():
            e0 = b * BLK
            pltpu.sync_copy(src_hbm.at[pl.ds(e0, BLK)], srcv)
            pltpu.sync_copy(dst_hbm.at[pl.ds(e0, BLK)], dstv)

            @pl.loop(0, BLK, step=16)
            def _(k):
                si = srcv[pl.ds(k, 16)]
                di = dstv[pl.ds(k, 16)]
                acc = jnp.zeros((16,), jnp.float32)
                for pv in (px_v, py_v, pz_v):
                    a = plsc.load_gather(pv, [si])
                    bb = plsc.load_gather(pv, [di])
                    d = a - bb
                    acc = acc + d * d
                d2v[pl.ds(k, 16)] = acc

            pltpu.sync_copy(d2v, d2_hbm.at[b])


@functools.partial(
    pl.kernel,
    out_type=jax.ShapeDtypeStruct((NC, NPAD, C), jnp.float32),
    mesh=_mesh,
    scratch_types=[
        pltpu.VMEM((2, BLK), jnp.int32),
        pltpu.VMEM((BLK, C), jnp.float32),
        pltpu.VMEM_SHARED((NPAD, C), jnp.float32),
        pltpu.SemaphoreType.DMA,
    ],
    compiler_params=_sc_params,
)
def _sc_deg(h_hbm, dst_hbm, out_hbm, dstv2, hv, acc_sh, ssc):
    cidx = lax.axis_index("c")
    sidx = lax.axis_index("s")
    w = _tile_id()

    @pl.loop(0, BLK)
    def _(j):
        @pl.loop(0, C, step=16)
        def _(k):
            hv[j, pl.ds(k, 16)] = jnp.zeros((16,), jnp.float32)

    r0 = sidx * ROWS_PER_SUB
    for j in range(ROWS_PER_SUB // ZCHUNK):
        pltpu.sync_copy(hv, acc_sh.at[pl.ds(r0 + j * ZCHUNK, ZCHUNK)])
    plsc.subcore_barrier()

    def scat(s):
        return pltpu.make_async_copy(hv, acc_sh.at[dstv2.at[s]], ssc)

    @pl.loop(0, MAX_BLK_PER_TILE, step=2)
    def _(ii):
        for t in range(2):
            i = ii + t
            b = w + i * NW

            @pl.when(b < NBLK)
            def _():
                e0 = b * BLK
                pltpu.sync_copy(dst_hbm.at[pl.ds(e0, BLK)], dstv2.at[t])

                @pl.when(i >= 1)
                def _():
                    scat(1 - t).wait()

                pltpu.sync_copy(h_hbm.at[pl.ds(e0, BLK)], hv)
                scat(t).start(add=True)

    nb = MAX_BLK_PER_TILE - jnp.where(w + (MAX_BLK_PER_TILE - 1) * NW < NBLK,
                                      0, 1)
    last_slot = (nb - 1) % 2

    @pl.when(last_slot == 0)
    def _():
        scat(0).wait()

    @pl.when(last_slot == 1)
    def _():
        scat(1).wait()

    plsc.subcore_barrier()
    pltpu.sync_copy(acc_sh.at[pl.ds(r0, ROWS_PER_SUB)],
                    out_hbm.at[cidx, pl.ds(r0, ROWS_PER_SUB)])


@functools.partial(
    pl.kernel,
    out_type=jax.ShapeDtypeStruct((NC, NPAD, C), jnp.float32),
    mesh=_mesh,
    scratch_types=[
        pltpu.VMEM((BLK,), jnp.int32),
        pltpu.VMEM((2, BLK), jnp.int32),
        pltpu.VMEM((BLK, C), jnp.float32),
        pltpu.VMEM((BLK, C), jnp.float32),
        pltpu.VMEM_SHARED((NPAD, C), jnp.float32),
        pltpu.SemaphoreType.DMA,
        pltpu.SemaphoreType.DMA,
    ],
    compiler_params=_sc_params,
)
def _sc_layer(x_hbm, r_hbm, src_hbm, dst_hbm, out_hbm,
              srcv, dstv2, xg, rv, acc_sh, sem, ssc):
    cidx = lax.axis_index("c")
    sidx = lax.axis_index("s")
    w = _tile_id()

    @pl.loop(0, BLK)
    def _(j):
        @pl.loop(0, C, step=16)
        def _(k):
            xg[j, pl.ds(k, 16)] = jnp.zeros((16,), jnp.float32)

    r0 = sidx * ROWS_PER_SUB
    for j in range(ROWS_PER_SUB // ZCHUNK):
        pltpu.sync_copy(xg, acc_sh.at[pl.ds(r0 + j * ZCHUNK, ZCHUNK)])
    plsc.subcore_barrier()

    def scat(s):
        return pltpu.make_async_copy(xg, acc_sh.at[dstv2.at[s]], ssc)

    @pl.loop(0, MAX_BLK_PER_TILE, step=2)
    def _(ii):
        for t in range(2):
            i = ii + t
            b = w + i * NW

            @pl.when(b < NBLK)
            def _():
                e0 = b * BLK
                pltpu.sync_copy(src_hbm.at[pl.ds(e0, BLK)], srcv)
                pltpu.sync_copy(dst_hbm.at[pl.ds(e0, BLK)], dstv2.at[t])

                @pl.when(i >= 1)
                def _():
                    scat(1 - t).wait()

                cp = pltpu.async_copy(x_hbm.at[srcv], xg, sem)
                pltpu.sync_copy(r_hbm.at[pl.ds(e0, BLK)], rv)
                cp.wait()

                @pl.loop(0, BLK)
                def _(j):
                    @pl.loop(0, C, step=16)
                    def _(k):
                        xg[j, pl.ds(k, 16)] = (
                            xg[j, pl.ds(k, 16)] * rv[j, pl.ds(k, 16)])

                scat(t).start(add=True)

    nb = MAX_BLK_PER_TILE - jnp.where(w + (MAX_BLK_PER_TILE - 1) * NW < NBLK,
                                      0, 1)
    last_slot = (nb - 1) % 2

    @pl.when(last_slot == 0)
    def _():
        scat(0).wait()

    @pl.when(last_slot == 1)
    def _():
        scat(1).wait()

    plsc.subcore_barrier()
    pltpu.sync_copy(acc_sh.at[pl.ds(r0, ROWS_PER_SUB)],
                    out_hbm.at[cidx, pl.ds(r0, ROWS_PER_SUB)])


def _enc_from_d2(d2b):
    dist = jnp.sqrt(d2b + 1e-12)
    delta = MAX_RADIUS / (BASIS - 1)
    coeff = -0.5 / (2.0 * delta) ** 2
    off = lax.broadcasted_iota(jnp.int32, (EB, BASIS), 1).astype(jnp.float32)
    off = off * delta
    return jnp.exp(coeff * (dist - off) ** 2)


def _edge_mlp_body(nmlp, d2_ref, w1_ref, b1_ref, w2_ref, b2_ref, *out_refs):
    enc = _enc_from_d2(d2_ref[...])
    for i in range(nmlp):
        t = jnp.dot(enc, w1_ref[i], preferred_element_type=jnp.float32)
        t = _silu(t + b1_ref[i][None, :])
        o = jnp.dot(t, w2_ref[i], preferred_element_type=jnp.float32)
        out_refs[i][...] = o + b2_ref[i][None, :]


def _tc_edge_mlp(nmlp, d2, w1s, b1s, w2s, b2s):
    grid = (E // EB,)
    outs = [jax.ShapeDtypeStruct((E, C), jnp.float32) for _ in range(nmlp)]
    return pl.pallas_call(
        functools.partial(_edge_mlp_body, nmlp),
        grid=grid,
        in_specs=[
            pl.BlockSpec((EB, 1), lambda i: (i, 0)),
            pl.BlockSpec((nmlp, C, C), lambda i: (0, 0, 0)),
            pl.BlockSpec((nmlp, C), lambda i: (0, 0)),
            pl.BlockSpec((nmlp, C, C), lambda i: (0, 0, 0)),
            pl.BlockSpec((nmlp, C), lambda i: (0, 0)),
        ],
        out_specs=[pl.BlockSpec((EB, C), lambda i: (i, 0))] * nmlp,
        out_shape=outs,
    )(d2, w1s, b1s, w2s, b2s)


def _node_init_body(an_ref, sph_ref, acc_ref, out_ref):
    ids = an_ref[...]
    oh = (ids == lax.broadcasted_iota(jnp.int32, (NB, C), 1))
    sph = jnp.dot(oh.astype(jnp.float32), sph_ref[...],
                  preferred_element_type=jnp.float32)
    acc = (acc_ref[0] + acc_ref[1]) * (1.0 / AVG_DEGREE)
    out_ref[...] = sph + acc


def _tc_node_init(an2, sphere_pad, accpair):
    return pl.pallas_call(
        _node_init_body,
        grid=(N // NB,),
        in_specs=[
            pl.BlockSpec((NB, 1), lambda i: (i, 0)),
            pl.BlockSpec((C, C), lambda i: (0, 0)),
            pl.BlockSpec((NC, NB, C), lambda i: (0, i, 0)),
        ],
        out_specs=pl.BlockSpec((NB, C), lambda i: (i, 0)),
        out_shape=jax.ShapeDtypeStruct((N, C), jnp.float32),
    )(an2, sphere_pad, accpair)


def _update_body(x_ref, acc_ref, wm_ref, wf1_ref, bf1_ref, wf2_ref, bf2_ref,
                 out_ref):
    p = acc_ref[0] + acc_ref[1]
    agg = jnp.dot(p, wm_ref[...], preferred_element_type=jnp.float32)
    y = x_ref[...] + agg * (1.0 / AVG_DEGREE)
    t = jnp.dot(y, wf1_ref[...], preferred_element_type=jnp.float32)
    t = _silu(t + bf1_ref[...])
    f = jnp.dot(t, wf2_ref[...], preferred_element_type=jnp.float32)
    out_ref[...] = y + f + bf2_ref[...]


def _tc_update(x, accpair, wm, wf1, bf1, wf2, bf2):
    return pl.pallas_call(
        _update_body,
        grid=(N // NB,),
        in_specs=[
            pl.BlockSpec((NB, C), lambda i: (i, 0)),
            pl.BlockSpec((NC, NB, C), lambda i: (0, i, 0)),
            pl.BlockSpec((C, C), lambda i: (0, 0)),
            pl.BlockSpec((C, H), lambda i: (0, 0)),
            pl.BlockSpec((1, H), lambda i: (0, 0)),
            pl.BlockSpec((H, C), lambda i: (0, 0)),
            pl.BlockSpec((1, C), lambda i: (0, 0)),
        ],
        out_specs=pl.BlockSpec((NB, C), lambda i: (i, 0)),
        out_shape=jax.ShapeDtypeStruct((N, C), jnp.float32),
    )(x, accpair, wm, wf1, bf1, wf2, bf2)


def _final_body(x_ref, batch_ref, ns_ref, we_ref, be_ref, out_ref):
    xb = x_ref[...]
    ms = jnp.mean(xb * xb, axis=1, keepdims=True)
    xn = xb / jnp.sqrt(ms + 1e-6) * ns_ref[...]
    ne = jnp.dot(xn, we_ref[...], preferred_element_type=jnp.float32)
    ne = ne + be_ref[...]
    oh = (batch_ref[...] == lax.broadcasted_iota(jnp.int32, (NB, NG), 1))
    contrib = jnp.sum(oh.astype(jnp.float32) * ne, axis=0, keepdims=True)

    @pl.when(pl.program_id(0) == 0)
    def _():
        out_ref[...] = jnp.zeros_like(out_ref)

    out_ref[...] += contrib * (1.0 / AVG_NUM_NODES)


def _tc_final(x, batch2, norm_scale2, we, be2):
    return pl.pallas_call(
        _final_body,
        grid=(N // NB,),
        in_specs=[
            pl.BlockSpec((NB, C), lambda i: (i, 0)),
            pl.BlockSpec((NB, 1), lambda i: (i, 0)),
            pl.BlockSpec((1, C), lambda i: (0, 0)),
            pl.BlockSpec((C, 1), lambda i: (0, 0)),
            pl.BlockSpec((1, 1), lambda i: (0, 0)),
        ],
        out_specs=pl.BlockSpec((1, NG), lambda i: (0, 0)),
        out_shape=jax.ShapeDtypeStruct((1, NG), jnp.float32),
    )(x, batch2, norm_scale2, we, be2)


def kernel(atomic_numbers, pos, edge_index, batch, natoms, sphere_table,
           Wdeg1, bdeg1, Wdeg2, bdeg2, Wd1, bd1, Wd2, bd2, Wmsg,
           Wf1, bf1, Wf2, bf2, norm_scale, We, be):
    src = edge_index[0]
    dst = edge_index[1]

    posf = jnp.asarray(pos, jnp.float32)
    d2 = _sc_d2(posf[:, 0], posf[:, 1], posf[:, 2], src, dst)

    d2c = d2.reshape(E, 1)
    (h,) = _tc_edge_mlp(1, d2c, Wdeg1[None], bdeg1[None], Wdeg2[None],
                        bdeg2[None])
    rs = _tc_edge_mlp(NL, d2c, Wd1, bd1, Wd2, bd2)

    accd = _sc_deg(h, dst)
    sphere_pad = jnp.zeros((C, C), jnp.float32).at[: sphere_table.shape[0]].set(
        sphere_table)
    x = _tc_node_init(atomic_numbers.reshape(N, 1), sphere_pad, accd)

    for i in range(NL):
        acci = _sc_layer(x, rs[i], src, dst)
        x = _tc_update(x, acci, Wmsg[i], Wf1[i], bf1[i].reshape(1, H),
                       Wf2[i], bf2[i].reshape(1, C))

    out = _tc_final(x, batch.reshape(N, 1), norm_scale.reshape(1, C),
                    We, be.reshape(1, 1))
    return out.reshape(NG)


# speedup vs baseline: 1.0020x; 1.0020x over previous
import dataclasses
import functools

import jax
import jax.numpy as jnp
from jax import lax
from jax.experimental import pallas as pl
from jax.experimental.pallas import tpu as pltpu
from jax.experimental.pallas import tpu_sc as plsc

N = 10000
E = 160000
C = 128
BASIS = 128
NL = 4
H = 128
NG = 128
AVG_DEGREE = 23.395238876342773
AVG_NUM_NODES = 77.81317
MAX_RADIUS = 12.0

NC = 2
NS = 16
NW = NC * NS
BLK = 128
NBLK = E // BLK
MAX_BLK_PER_TILE = (NBLK + NW - 1) // NW
NPAD = 10240
ROWS_PER_SUB = NPAD // NS
ZCHUNK = 128

EB = 2000
NB = 1000

_mesh = plsc.VectorSubcoreMesh(core_axis_name="c", subcore_axis_name="s")

_sc_params = pltpu.CompilerParams()
if "needs_layout_passes" in pltpu.CompilerParams.__dataclass_fields__:
    _sc_params = dataclasses.replace(_sc_params, needs_layout_passes=False)


def _tile_id():
    return lax.axis_index("s") * NC + lax.axis_index("c")


def _silu(t):
    return t * (1.0 / (1.0 + jnp.exp(-t)))


@functools.partial(
    pl.kernel,
    out_type=jax.ShapeDtypeStruct((NBLK, BLK), jnp.float32),
    mesh=_mesh,
    scratch_types=[
        pltpu.VMEM((N,), jnp.float32),
        pltpu.VMEM((N,), jnp.float32),
        pltpu.VMEM((N,), jnp.float32),
        pltpu.VMEM((BLK,), jnp.int32),
        pltpu.VMEM((BLK,), jnp.int32),
        pltpu.VMEM((BLK,), jnp.float32),
    ],
    compiler_params=_sc_params,
)
def _sc_d2(px_hbm, py_hbm, pz_hbm, src_hbm, dst_hbm, d2_hbm,
           px_v, py_v, pz_v, srcv, dstv, d2v):
    w = _tile_id()
    pltpu.sync_copy(px_hbm, px_v)
    pltpu.sync_copy(py_hbm, py_v)
    pltpu.sync_copy(pz_hbm, pz_v)

    @pl.loop(0, MAX_BLK_PER_TILE)
    def _(i):
        b = w + i * NW

        @pl.when(b < NBLK)
        def _():
            e0 = b * BLK
            pltpu.sync_copy(src_hbm.at[pl.ds(e0, BLK)], srcv)
            pltpu.sync_copy(dst_hbm.at[pl.ds(e0, BLK)], dstv)

            @pl.loop(0, BLK, step=16)
            def _(k):
                si = srcv[pl.ds(k, 16)]
                di = dstv[pl.ds(k, 16)]
                acc = jnp.zeros((16,), jnp.float32)
                for pv in (px_v, py_v, pz_v):
                    a = plsc.load_gather(pv, [si])
                    bb = plsc.load_gather(pv, [di])
                    d = a - bb
                    acc = acc + d * d
                d2v[pl.ds(k, 16)] = acc

            pltpu.sync_copy(d2v, d2_hbm.at[b])


@functools.partial(
    pl.kernel,
    out_type=jax.ShapeDtypeStruct((NC, NPAD, C), jnp.float32),
    mesh=_mesh,
    scratch_types=[
        pltpu.VMEM((BLK,), jnp.int32),
        pltpu.VMEM((BLK, C), jnp.float32),
        pltpu.VMEM_SHARED((NPAD, C), jnp.float32),
    ],
    compiler_params=_sc_params,
)
def _sc_deg(h_hbm, dst_hbm, out_hbm, dstv, hv, acc_sh):
    cidx = lax.axis_index("c")
    sidx = lax.axis_index("s")
    w = _tile_id()

    @pl.loop(0, BLK)
    def _(j):
        @pl.loop(0, C, step=16)
        def _(k):
            hv[j, pl.ds(k, 16)] = jnp.zeros((16,), jnp.float32)

    r0 = sidx * ROWS_PER_SUB
    for j in range(ROWS_PER_SUB // ZCHUNK):
        pltpu.sync_copy(hv, acc_sh.at[pl.ds(r0 + j * ZCHUNK, ZCHUNK)])
    plsc.subcore_barrier()

    @pl.loop(0, MAX_BLK_PER_TILE)
    def _(i):
        b = w + i * NW

        @pl.when(b < NBLK)
        def _():
            e0 = b * BLK
            pltpu.sync_copy(dst_hbm.at[pl.ds(e0, BLK)], dstv)
            pltpu.sync_copy(h_hbm.at[pl.ds(e0, BLK)], hv)
            pltpu.sync_copy(hv, acc_sh.at[dstv], add=True)

    plsc.subcore_barrier()
    pltpu.sync_copy(acc_sh.at[pl.ds(r0, ROWS_PER_SUB)],
                    out_hbm.at[cidx, pl.ds(r0, ROWS_PER_SUB)])


@functools.partial(
    pl.kernel,
    out_type=jax.ShapeDtypeStruct((NC, NPAD, C), jnp.float32),
    mesh=_mesh,
    scratch_types=[
        pltpu.VMEM((BLK,), jnp.int32),
        pltpu.VMEM((2, BLK), jnp.int32),
        pltpu.VMEM((BLK, C), jnp.float32),
        pltpu.VMEM((BLK, C), jnp.float32),
        pltpu.VMEM_SHARED((NPAD, C), jnp.float32),
        pltpu.SemaphoreType.DMA,
        pltpu.SemaphoreType.DMA,
    ],
    compiler_params=_sc_params,
)
def _sc_layer(x_hbm, r_hbm, src_hbm, dst_hbm, out_hbm,
              srcv, dstv2, xg, rv, acc_sh, sem, ssc):
    cidx = lax.axis_index("c")
    sidx = lax.axis_index("s")
    w = _tile_id()

    @pl.loop(0, BLK)
    def _(j):
        @pl.loop(0, C, step=16)
        def _(k):
            xg[j, pl.ds(k, 16)] = jnp.zeros((16,), jnp.float32)

    r0 = sidx * ROWS_PER_SUB
    for j in range(ROWS_PER_SUB // ZCHUNK):
        pltpu.sync_copy(xg, acc_sh.at[pl.ds(r0 + j * ZCHUNK, ZCHUNK)])
    plsc.subcore_barrier()

    def scat(s):
        return pltpu.make_async_copy(xg, acc_sh.at[dstv2.at[s]], ssc)

    @pl.loop(0, MAX_BLK_PER_TILE, step=2)
    def _(ii):
        for t in range(2):
            i = ii + t
            b = w + i * NW

            @pl.when(b < NBLK)
            def _():
                e0 = b * BLK
                pltpu.sync_copy(src_hbm.at[pl.ds(e0, BLK)], srcv)
                pltpu.sync_copy(dst_hbm.at[pl.ds(e0, BLK)], dstv2.at[t])

                @pl.when(i >= 1)
                def _():
                    scat(1 - t).wait()

                cp = pltpu.async_copy(x_hbm.at[srcv], xg, sem)
                pltpu.sync_copy(r_hbm.at[pl.ds(e0, BLK)], rv)
                cp.wait()

                @pl.loop(0, BLK)
                def _(j):
                    @pl.loop(0, C, step=16)
                    def _(k):
                        xg[j, pl.ds(k, 16)] = (
                            xg[j, pl.ds(k, 16)] * rv[j, pl.ds(k, 16)])

                scat(t).start(add=True)

    nb = MAX_BLK_PER_TILE - jnp.where(w + (MAX_BLK_PER_TILE - 1) * NW < NBLK,
                                      0, 1)
    last_slot = (nb - 1) % 2

    @pl.when(last_slot == 0)
    def _():
        scat(0).wait()

    @pl.when(last_slot == 1)
    def _():
        scat(1).wait()

    plsc.subcore_barrier()
    pltpu.sync_copy(acc_sh.at[pl.ds(r0, ROWS_PER_SUB)],
                    out_hbm.at[cidx, pl.ds(r0, ROWS_PER_SUB)])


def _enc_from_d2(d2b):
    dist = jnp.sqrt(d2b + 1e-12)
    delta = MAX_RADIUS / (BASIS - 1)
    coeff = -0.5 / (2.0 * delta) ** 2
    off = lax.broadcasted_iota(jnp.int32, (EB, BASIS), 1).astype(jnp.float32)
    off = off * delta
    return jnp.exp(coeff * (dist - off) ** 2)


def _edge_mlp_body(nmlp, d2_ref, w1_ref, b1_ref, w2_ref, b2_ref, *out_refs):
    enc = _enc_from_d2(d2_ref[...])
    for i in range(nmlp):
        t = jnp.dot(enc, w1_ref[i], preferred_element_type=jnp.float32)
        t = _silu(t + b1_ref[i][None, :])
        o = jnp.dot(t, w2_ref[i], preferred_element_type=jnp.float32)
        out_refs[i][...] = o + b2_ref[i][None, :]


def _tc_edge_mlp(nmlp, d2, w1s, b1s, w2s, b2s):
    grid = (E // EB,)
    outs = [jax.ShapeDtypeStruct((E, C), jnp.float32) for _ in range(nmlp)]
    return pl.pallas_call(
        functools.partial(_edge_mlp_body, nmlp),
        grid=grid,
        in_specs=[
            pl.BlockSpec((EB, 1), lambda i: (i, 0)),
            pl.BlockSpec((nmlp, C, C), lambda i: (0, 0, 0)),
            pl.BlockSpec((nmlp, C), lambda i: (0, 0)),
            pl.BlockSpec((nmlp, C, C), lambda i: (0, 0, 0)),
            pl.BlockSpec((nmlp, C), lambda i: (0, 0)),
        ],
        out_specs=[pl.BlockSpec((EB, C), lambda i: (i, 0))] * nmlp,
        out_shape=outs,
    )(d2, w1s, b1s, w2s, b2s)


def _node_init_body(an_ref, sph_ref, acc_ref, out_ref):
    ids = an_ref[...]
    oh = (ids == lax.broadcasted_iota(jnp.int32, (NB, C), 1))
    sph = jnp.dot(oh.astype(jnp.float32), sph_ref[...],
                  preferred_element_type=jnp.float32)
    acc = (acc_ref[0] + acc_ref[1]) * (1.0 / AVG_DEGREE)
    out_ref[...] = sph + acc


def _tc_node_init(an2, sphere_pad, accpair):
    return pl.pallas_call(
        _node_init_body,
        grid=(N // NB,),
        in_specs=[
            pl.BlockSpec((NB, 1), lambda i: (i, 0)),
            pl.BlockSpec((C, C), lambda i: (0, 0)),
            pl.BlockSpec((NC, NB, C), lambda i: (0, i, 0)),
        ],
        out_specs=pl.BlockSpec((NB, C), lambda i: (i, 0)),
        out_shape=jax.ShapeDtypeStruct((N, C), jnp.float32),
    )(an2, sphere_pad, accpair)


def _update_body(x_ref, acc_ref, wm_ref, wf1_ref, bf1_ref, wf2_ref, bf2_ref,
                 out_ref):
    p = acc_ref[0] + acc_ref[1]
    agg = jnp.dot(p, wm_ref[...], preferred_element_type=jnp.float32)
    y = x_ref[...] + agg * (1.0 / AVG_DEGREE)
    t = jnp.dot(y, wf1_ref[...], preferred_element_type=jnp.float32)
    t = _silu(t + bf1_ref[...])
    f = jnp.dot(t, wf2_ref[...], preferred_element_type=jnp.float32)
    out_ref[...] = y + f + bf2_ref[...]


def _tc_update(x, accpair, wm, wf1, bf1, wf2, bf2):
    return pl.pallas_call(
        _update_body,
        grid=(N // NB,),
        in_specs=[
            pl.BlockSpec((NB, C), lambda i: (i, 0)),
            pl.BlockSpec((NC, NB, C), lambda i: (0, i, 0)),
            pl.BlockSpec((C, C), lambda i: (0, 0)),
            pl.BlockSpec((C, H), lambda i: (0, 0)),
            pl.BlockSpec((1, H), lambda i: (0, 0)),
            pl.BlockSpec((H, C), lambda i: (0, 0)),
            pl.BlockSpec((1, C), lambda i: (0, 0)),
        ],
        out_specs=pl.BlockSpec((NB, C), lambda i: (i, 0)),
        out_shape=jax.ShapeDtypeStruct((N, C), jnp.float32),
    )(x, accpair, wm, wf1, bf1, wf2, bf2)


def _final_body(x_ref, batch_ref, ns_ref, we_ref, be_ref, out_ref):
    xb = x_ref[...]
    ms = jnp.mean(xb * xb, axis=1, keepdims=True)
    xn = xb / jnp.sqrt(ms + 1e-6) * ns_ref[...]
    ne = jnp.dot(xn, we_ref[...], preferred_element_type=jnp.float32)
    ne = ne + be_ref[...]
    oh = (batch_ref[...] == lax.broadcasted_iota(jnp.int32, (NB, NG), 1))
    contrib = jnp.sum(oh.astype(jnp.float32) * ne, axis=0, keepdims=True)

    @pl.when(pl.program_id(0) == 0)
    def _():
        out_ref[...] = jnp.zeros_like(out_ref)

    out_ref[...] += contrib * (1.0 / AVG_NUM_NODES)


def _tc_final(x, batch2, norm_scale2, we, be2):
    return pl.pallas_call(
        _final_body,
        grid=(N // NB,),
        in_specs=[
            pl.BlockSpec((NB, C), lambda i: (i, 0)),
            pl.BlockSpec((NB, 1), lambda i: (i, 0)),
            pl.BlockSpec((1, C), lambda i: (0, 0)),
            pl.BlockSpec((C, 1), lambda i: (0, 0)),
            pl.BlockSpec((1, 1), lambda i: (0, 0)),
        ],
        out_specs=pl.BlockSpec((1, NG), lambda i: (0, 0)),
        out_shape=jax.ShapeDtypeStruct((1, NG), jnp.float32),
    )(x, batch2, norm_scale2, we, be2)


def kernel(atomic_numbers, pos, edge_index, batch, natoms, sphere_table,
           Wdeg1, bdeg1, Wdeg2, bdeg2, Wd1, bd1, Wd2, bd2, Wmsg,
           Wf1, bf1, Wf2, bf2, norm_scale, We, be):
    src = edge_index[0]
    dst = edge_index[1]

    posf = jnp.asarray(pos, jnp.float32)
    d2 = _sc_d2(posf[:, 0], posf[:, 1], posf[:, 2], src, dst)

    d2c = d2.reshape(E, 1)
    (h,) = _tc_edge_mlp(1, d2c, Wdeg1[None], bdeg1[None], Wdeg2[None],
                        bdeg2[None])
    rs = _tc_edge_mlp(NL, d2c, Wd1, bd1, Wd2, bd2)

    accd = _sc_deg(h, dst)
    sphere_pad = jnp.zeros((C, C), jnp.float32).at[: sphere_table.shape[0]].set(
        sphere_table)
    x = _tc_node_init(atomic_numbers.reshape(N, 1), sphere_pad, accd)

    for i in range(NL):
        acci = _sc_layer(x, rs[i], src, dst)
        x = _tc_update(x, acci, Wmsg[i], Wf1[i], bf1[i].reshape(1, H),
                       Wf2[i], bf2[i].reshape(1, C))

    out = _tc_final(x, batch.reshape(N, 1), norm_scale.reshape(1, C),
                    We, be.reshape(1, 1))
    return out.reshape(NG)
